# Initial kernel scaffold; baseline (speedup 1.0000x reference)
import functools

import jax
import jax.numpy as jnp
from jax import lax
from jax.experimental import pallas as pl
from jax.experimental.pallas import tpu as pltpu
from jax.experimental.pallas import tpu_sc as plsc

N = 10000
E = 320000
D = 128
H = 128
C = 40
CP = 48
BN_EPS = 1e-5

NC = 2
NS = 16
K = 80
EPT = E // (NC * NS)
NCHUNK = EPT // K
RPT = N // NS
ZR = 125


def _zero_fill(ref, nrows, width):
    zeros16 = jnp.zeros((16,), jnp.float32)

    @pl.loop(0, nrows)
    def _(r):
        for g in range(width // 16):
            ref[r, pl.ds(g * 16, 16)] = zeros16


def _sc_degrees(src, dst):
    mesh = plsc.VectorSubcoreMesh(core_axis_name="c", subcore_axis_name="s")

    @functools.partial(
        pl.kernel,
        out_type=jax.ShapeDtypeStruct((NC, 2, N, 16), jnp.float32),
        mesh=mesh,
        scratch_types=[
            pltpu.VMEM((K,), jnp.int32),
            pltpu.VMEM((K,), jnp.int32),
            pltpu.VMEM((K, 16), jnp.float32),
            pltpu.VMEM((ZR, 16), jnp.float32),
            pltpu.VMEM_SHARED((N, 16), jnp.float32),
            pltpu.VMEM_SHARED((N, 16), jnp.float32),
        ],
    )
    def deg_kernel(src_hbm, dst_hbm, out_hbm, idx_s, idx_d, ones_v, zbuf,
                   dout_sh, din_sh):
        c = lax.axis_index("c")
        s = lax.axis_index("s")

        @pl.loop(0, K)
        def _(r):
            ones_v[r, pl.ds(0, 16)] = jnp.ones((16,), jnp.float32)

        _zero_fill(zbuf, ZR, 16)
        for j in range(RPT // ZR):
            base = s * RPT + j * ZR
            pltpu.sync_copy(zbuf, dout_sh.at[pl.ds(base, ZR)])
            pltpu.sync_copy(zbuf, din_sh.at[pl.ds(base, ZR)])
        plsc.subcore_barrier()

        wid = c * NS + s
        ebase = wid * EPT

        @pl.loop(0, NCHUNK)
        def _(i):
            e0 = ebase + i * K
            pltpu.sync_copy(src_hbm.at[pl.ds(e0, K)], idx_s)
            pltpu.sync_copy(dst_hbm.at[pl.ds(e0, K)], idx_d)
            pltpu.sync_copy(ones_v, dout_sh.at[idx_s], add=True)
            pltpu.sync_copy(ones_v, din_sh.at[idx_d], add=True)

        plsc.subcore_barrier()
        rbase = s * RPT
        pltpu.sync_copy(dout_sh.at[pl.ds(rbase, RPT)],
                        out_hbm.at[c, 0, pl.ds(rbase, RPT)])
        pltpu.sync_copy(din_sh.at[pl.ds(rbase, RPT)],
                        out_hbm.at[c, 1, pl.ds(rbase, RPT)])

    return deg_kernel(src, dst)


def _sc_aggregate(h, src, dst, width):
    mesh = plsc.VectorSubcoreMesh(core_axis_name="c", subcore_axis_name="s")

    @functools.partial(
        pl.kernel,
        out_type=jax.ShapeDtypeStruct((NC, N, width), jnp.float32),
        mesh=mesh,
        scratch_types=[
            pltpu.VMEM((K,), jnp.int32),
            pltpu.VMEM((K,), jnp.int32),
            pltpu.VMEM((K, width), jnp.float32),
            pltpu.VMEM((ZR, width), jnp.float32),
            pltpu.VMEM_SHARED((N, width), jnp.float32),
            pltpu.SemaphoreType.DMA,
        ],
    )
    def agg_kernel(h_hbm, src_hbm, dst_hbm, out_hbm, idx_s, idx_d, rows,
                   zbuf, acc_sh, sem):
        c = lax.axis_index("c")
        s = lax.axis_index("s")

        _zero_fill(zbuf, ZR, width)
        for j in range(RPT // ZR):
            base = s * RPT + j * ZR
            pltpu.sync_copy(zbuf, acc_sh.at[pl.ds(base, ZR)])
        plsc.subcore_barrier()

        wid = c * NS + s
        ebase = wid * EPT

        @pl.loop(0, NCHUNK)
        def _(i):
            e0 = ebase + i * K
            pltpu.sync_copy(src_hbm.at[pl.ds(e0, K)], idx_s)
            pltpu.sync_copy(dst_hbm.at[pl.ds(e0, K)], idx_d)
            pltpu.async_copy(h_hbm.at[idx_s], rows, sem).wait()
            pltpu.sync_copy(rows, acc_sh.at[idx_d], add=True)

        plsc.subcore_barrier()
        rbase = s * RPT
        pltpu.sync_copy(acc_sh.at[pl.ds(rbase, RPT)],
                        out_hbm.at[c, pl.ds(rbase, RPT)])

    return agg_kernel(h, src, dst)


def _norm_from_deg(deg):
    return jnp.where(deg > 0.0, lax.rsqrt(jnp.maximum(deg, 1e-30)), 0.0)


BN_ROWS = 1000


def _tc_layer1_pre(x, W1, deg_part):
    def body(x_ref, w_ref, dp_ref, h_ref, ds_ref):
        dp = dp_ref[...]
        dsum = dp[0] + dp[1]
        ds_ref[...] = dsum
        nout = _norm_from_deg(dsum[0][:, 0:1])
        h = jnp.dot(x_ref[...], w_ref[...], preferred_element_type=jnp.float32)
        h_ref[...] = h * nout

    return pl.pallas_call(
        body,
        grid=(N // BN_ROWS,),
        in_specs=[
            pl.BlockSpec((BN_ROWS, D), lambda i: (i, 0)),
            pl.BlockSpec((D, H), lambda i: (0, 0)),
            pl.BlockSpec((NC, 2, BN_ROWS, 16), lambda i: (0, 0, i, 0)),
        ],
        out_specs=[
            pl.BlockSpec((BN_ROWS, H), lambda i: (i, 0)),
            pl.BlockSpec((2, BN_ROWS, 16), lambda i: (0, i, 0)),
        ],
        out_shape=[
            jax.ShapeDtypeStruct((N, H), jnp.float32),
            jax.ShapeDtypeStruct((2, N, 16), jnp.float32),
        ],
    )(x, W1, deg_part)


def _tc_combine_stats(agg_part, degsum, b1):
    def body(a_ref, ds_ref, b_ref, t_ref, st_ref):
        i = pl.program_id(0)
        nin = _norm_from_deg(ds_ref[1][:, 0:1])
        t = (a_ref[0] + a_ref[1]) * nin + b_ref[...]
        t_ref[...] = t

        @pl.when(i == 0)
        def _():
            st_ref[...] = jnp.zeros_like(st_ref)

        st_ref[0:1, :] += jnp.sum(t, axis=0, keepdims=True)
        st_ref[1:2, :] += jnp.sum(t * t, axis=0, keepdims=True)

    return pl.pallas_call(
        body,
        grid=(N // BN_ROWS,),
        in_specs=[
            pl.BlockSpec((NC, BN_ROWS, H), lambda i: (0, i, 0)),
            pl.BlockSpec((2, BN_ROWS, 16), lambda i: (0, i, 0)),
            pl.BlockSpec((1, H), lambda i: (0, 0)),
        ],
        out_specs=[
            pl.BlockSpec((BN_ROWS, H), lambda i: (i, 0)),
            pl.BlockSpec((8, H), lambda i: (0, 0)),
        ],
        out_shape=[
            jax.ShapeDtypeStruct((N, H), jnp.float32),
            jax.ShapeDtypeStruct((8, H), jnp.float32),
        ],
    )(agg_part, degsum, b1)


def _tc_bn_relu_mm(t, stats, gamma1, beta1, W2p, degsum):
    def body(t_ref, st_ref, g_ref, be_ref, w_ref, ds_ref, o_ref):
        inv_n = 1.0 / N
        mean = st_ref[0:1, :] * inv_n
        var = st_ref[1:2, :] * inv_n - mean * mean
        scale = g_ref[...] * lax.rsqrt(var + BN_EPS)
        z = (t_ref[...] - mean) * scale + be_ref[...]
        z = jnp.maximum(z, 0.0)
        nout = _norm_from_deg(ds_ref[0][:, 0:1])
        h2 = jnp.dot(z, w_ref[...], preferred_element_type=jnp.float32)
        o_ref[...] = h2 * nout

    return pl.pallas_call(
        body,
        grid=(N // BN_ROWS,),
        in_specs=[
            pl.BlockSpec((BN_ROWS, H), lambda i: (i, 0)),
            pl.BlockSpec((8, H), lambda i: (0, 0)),
            pl.BlockSpec((1, H), lambda i: (0, 0)),
            pl.BlockSpec((1, H), lambda i: (0, 0)),
            pl.BlockSpec((H, CP), lambda i: (0, 0)),
            pl.BlockSpec((2, BN_ROWS, 16), lambda i: (0, i, 0)),
        ],
        out_specs=pl.BlockSpec((BN_ROWS, CP), lambda i: (i, 0)),
        out_shape=jax.ShapeDtypeStruct((N, CP), jnp.float32),
    )(t, stats, gamma1, beta1, W2p, degsum)


def _tc_final(agg_part, degsum, b2p):
    def body(a_ref, ds_ref, b_ref, o_ref):
        nin = _norm_from_deg(ds_ref[1][:, 0:1])
        u = (a_ref[0] + a_ref[1]) * nin + b_ref[...]
        col = lax.broadcasted_iota(jnp.int32, (BN_ROWS, CP), 1)
        valid = col < C
        um = jnp.where(valid, u, -1e30)
        m = jnp.max(um, axis=1, keepdims=True)
        e = jnp.where(valid, jnp.exp(um - m), 0.0)
        ssum = jnp.sum(e, axis=1, keepdims=True)
        o_ref[...] = u - m - jnp.log(ssum)

    return pl.pallas_call(
        body,
        grid=(N // BN_ROWS,),
        in_specs=[
            pl.BlockSpec((NC, BN_ROWS, CP), lambda i: (0, i, 0)),
            pl.BlockSpec((2, BN_ROWS, 16), lambda i: (0, i, 0)),
            pl.BlockSpec((1, CP), lambda i: (0, 0)),
        ],
        out_specs=pl.BlockSpec((BN_ROWS, CP), lambda i: (i, 0)),
        out_shape=jax.ShapeDtypeStruct((N, CP), jnp.float32),
    )(agg_part, degsum, b2p)


def kernel(x, edge_index, W1, b1, gamma1, beta1, W2, b2):
    src = edge_index[0]
    dst = edge_index[1]

    deg_part = _sc_degrees(src, dst)
    h1, degsum = _tc_layer1_pre(x, W1, deg_part)
    agg1 = _sc_aggregate(h1, src, dst, H)
    t, stats = _tc_combine_stats(agg1, degsum, b1.reshape(1, H))

    W2p = jnp.pad(W2, ((0, 0), (0, CP - C)))
    b2p = jnp.pad(b2, (0, CP - C)).reshape(1, CP)
    h2 = _tc_bn_relu_mm(t, stats, gamma1.reshape(1, H), beta1.reshape(1, H),
                        W2p, degsum)
    agg2 = _sc_aggregate(h2, src, dst, CP)
    out = _tc_final(agg2, degsum, b2p)
    return out[:, :C]


# speedup vs baseline: 4.9495x; 4.9495x over previous
import functools

import jax
import jax.numpy as jnp
from jax import lax
from jax.experimental import pallas as pl
from jax.experimental.pallas import tpu as pltpu
from jax.experimental.pallas import tpu_sc as plsc

N = 10000
E = 320000
D = 128
H = 128
C = 40
CP = 48
BN_EPS = 1e-5

NC = 2
NS = 16
K = 80
EPT = E // (NC * NS)
NCHUNK = EPT // K
RPT = 624
TAIL = N - NS * RPT
ZR = 104


def _zero_fill(ref, nrows, width):
    zeros16 = jnp.zeros((16,), jnp.float32)

    @pl.loop(0, nrows)
    def _(r):
        for g in range(width // 16):
            ref[r, pl.ds(g * 16, 16)] = zeros16


def _sc_degrees(src, dst):
    mesh = plsc.VectorSubcoreMesh(core_axis_name="c", subcore_axis_name="s")

    @functools.partial(
        pl.kernel,
        out_type=jax.ShapeDtypeStruct((NC, 2, N, 16), jnp.float32),
        mesh=mesh,
        compiler_params=pltpu.CompilerParams(use_tc_tiling_on_sc=False),
        scratch_types=[
            pltpu.VMEM((K,), jnp.int32),
            pltpu.VMEM((K,), jnp.int32),
            pltpu.VMEM((K, 16), jnp.float32),
            pltpu.VMEM((ZR, 16), jnp.float32),
            pltpu.VMEM_SHARED((N, 16), jnp.float32),
            pltpu.VMEM_SHARED((N, 16), jnp.float32),
        ],
    )
    def deg_kernel(src_hbm, dst_hbm, out_hbm, idx_s, idx_d, ones_v, zbuf,
                   dout_sh, din_sh):
        c = lax.axis_index("c")
        s = lax.axis_index("s")

        @pl.loop(0, K)
        def _(r):
            ones_v[r, pl.ds(0, 16)] = jnp.ones((16,), jnp.float32)

        _zero_fill(zbuf, ZR, 16)
        for j in range(RPT // ZR):
            base = s * RPT + j * ZR
            pltpu.sync_copy(zbuf, dout_sh.at[pl.ds(base, ZR)])
            pltpu.sync_copy(zbuf, din_sh.at[pl.ds(base, ZR)])

        @pl.when(s == 0)
        def _():
            pltpu.sync_copy(zbuf.at[pl.ds(0, TAIL)],
                            dout_sh.at[pl.ds(NS * RPT, TAIL)])
            pltpu.sync_copy(zbuf.at[pl.ds(0, TAIL)],
                            din_sh.at[pl.ds(NS * RPT, TAIL)])

        plsc.subcore_barrier()

        wid = c * NS + s
        ebase = wid * EPT

        @pl.loop(0, NCHUNK)
        def _(i):
            e0 = ebase + i * K
            pltpu.sync_copy(src_hbm.at[pl.ds(e0, K)], idx_s)
            pltpu.sync_copy(dst_hbm.at[pl.ds(e0, K)], idx_d)
            pltpu.sync_copy(ones_v, dout_sh.at[idx_s], add=True)
            pltpu.sync_copy(ones_v, din_sh.at[idx_d], add=True)

        plsc.subcore_barrier()
        rbase = s * RPT
        pltpu.sync_copy(dout_sh.at[pl.ds(rbase, RPT)],
                        out_hbm.at[c, 0, pl.ds(rbase, RPT)])
        pltpu.sync_copy(din_sh.at[pl.ds(rbase, RPT)],
                        out_hbm.at[c, 1, pl.ds(rbase, RPT)])

        @pl.when(s == 0)
        def _():
            pltpu.sync_copy(dout_sh.at[pl.ds(NS * RPT, TAIL)],
                            out_hbm.at[c, 0, pl.ds(NS * RPT, TAIL)])
            pltpu.sync_copy(din_sh.at[pl.ds(NS * RPT, TAIL)],
                            out_hbm.at[c, 1, pl.ds(NS * RPT, TAIL)])

    return deg_kernel(src, dst)


def _sc_aggregate(h, src, dst, width):
    mesh = plsc.VectorSubcoreMesh(core_axis_name="c", subcore_axis_name="s")

    @functools.partial(
        pl.kernel,
        out_type=jax.ShapeDtypeStruct((NC, N, width), jnp.float32),
        mesh=mesh,
        compiler_params=pltpu.CompilerParams(use_tc_tiling_on_sc=False),
        scratch_types=[
            pltpu.VMEM((K,), jnp.int32),
            pltpu.VMEM((K,), jnp.int32),
            pltpu.VMEM((K, width), jnp.float32),
            pltpu.VMEM((ZR, width), jnp.float32),
            pltpu.VMEM_SHARED((N, width), jnp.float32),
            pltpu.SemaphoreType.DMA,
        ],
    )
    def agg_kernel(h_hbm, src_hbm, dst_hbm, out_hbm, idx_s, idx_d, rows,
                   zbuf, acc_sh, sem):
        c = lax.axis_index("c")
        s = lax.axis_index("s")

        _zero_fill(zbuf, ZR, width)
        for j in range(RPT // ZR):
            base = s * RPT + j * ZR
            pltpu.sync_copy(zbuf, acc_sh.at[pl.ds(base, ZR)])

        @pl.when(s == 0)
        def _():
            pltpu.sync_copy(zbuf.at[pl.ds(0, TAIL)],
                            acc_sh.at[pl.ds(NS * RPT, TAIL)])

        plsc.subcore_barrier()

        wid = c * NS + s
        ebase = wid * EPT

        @pl.loop(0, NCHUNK)
        def _(i):
            e0 = ebase + i * K
            pltpu.sync_copy(src_hbm.at[pl.ds(e0, K)], idx_s)
            pltpu.sync_copy(dst_hbm.at[pl.ds(e0, K)], idx_d)
            pltpu.async_copy(h_hbm.at[idx_s], rows, sem).wait()
            pltpu.sync_copy(rows, acc_sh.at[idx_d], add=True)

        plsc.subcore_barrier()
        rbase = s * RPT
        pltpu.sync_copy(acc_sh.at[pl.ds(rbase, RPT)],
                        out_hbm.at[c, pl.ds(rbase, RPT)])

        @pl.when(s == 0)
        def _():
            pltpu.sync_copy(acc_sh.at[pl.ds(NS * RPT, TAIL)],
                            out_hbm.at[c, pl.ds(NS * RPT, TAIL)])

    return agg_kernel(h, src, dst)


def _norm_from_deg(deg):
    return jnp.where(deg > 0.0, lax.rsqrt(jnp.maximum(deg, 1e-30)), 0.0)


BN_ROWS = 1000


def _tc_layer1_pre(x, W1, deg_part):
    def body(x_ref, w_ref, dp_ref, h_ref, ds_ref):
        dp = dp_ref[...]
        dsum = dp[0] + dp[1]
        ds_ref[...] = dsum
        nout = _norm_from_deg(dsum[0][:, 0:1])
        h = jnp.dot(x_ref[...], w_ref[...], preferred_element_type=jnp.float32)
        h_ref[...] = h * nout

    return pl.pallas_call(
        body,
        grid=(N // BN_ROWS,),
        in_specs=[
            pl.BlockSpec((BN_ROWS, D), lambda i: (i, 0)),
            pl.BlockSpec((D, H), lambda i: (0, 0)),
            pl.BlockSpec((NC, 2, BN_ROWS, 16), lambda i: (0, 0, i, 0)),
        ],
        out_specs=[
            pl.BlockSpec((BN_ROWS, H), lambda i: (i, 0)),
            pl.BlockSpec((2, BN_ROWS, 16), lambda i: (0, i, 0)),
        ],
        out_shape=[
            jax.ShapeDtypeStruct((N, H), jnp.float32),
            jax.ShapeDtypeStruct((2, N, 16), jnp.float32),
        ],
    )(x, W1, deg_part)


def _tc_combine_stats(agg_part, degsum, b1):
    def body(a_ref, ds_ref, b_ref, t_ref, st_ref):
        i = pl.program_id(0)
        nin = _norm_from_deg(ds_ref[1][:, 0:1])
        t = (a_ref[0] + a_ref[1]) * nin + b_ref[...]
        t_ref[...] = t

        @pl.when(i == 0)
        def _():
            st_ref[...] = jnp.zeros_like(st_ref)

        st_ref[0:1, :] += jnp.sum(t, axis=0, keepdims=True)
        st_ref[1:2, :] += jnp.sum(t * t, axis=0, keepdims=True)

    return pl.pallas_call(
        body,
        grid=(N // BN_ROWS,),
        in_specs=[
            pl.BlockSpec((NC, BN_ROWS, H), lambda i: (0, i, 0)),
            pl.BlockSpec((2, BN_ROWS, 16), lambda i: (0, i, 0)),
            pl.BlockSpec((1, H), lambda i: (0, 0)),
        ],
        out_specs=[
            pl.BlockSpec((BN_ROWS, H), lambda i: (i, 0)),
            pl.BlockSpec((8, H), lambda i: (0, 0)),
        ],
        out_shape=[
            jax.ShapeDtypeStruct((N, H), jnp.float32),
            jax.ShapeDtypeStruct((8, H), jnp.float32),
        ],
    )(agg_part, degsum, b1)


def _tc_bn_relu_mm(t, stats, gamma1, beta1, W2p, degsum):
    def body(t_ref, st_ref, g_ref, be_ref, w_ref, ds_ref, o_ref):
        inv_n = 1.0 / N
        mean = st_ref[0:1, :] * inv_n
        var = st_ref[1:2, :] * inv_n - mean * mean
        scale = g_ref[...] * lax.rsqrt(var + BN_EPS)
        z = (t_ref[...] - mean) * scale + be_ref[...]
        z = jnp.maximum(z, 0.0)
        nout = _norm_from_deg(ds_ref[0][:, 0:1])
        h2 = jnp.dot(z, w_ref[...], preferred_element_type=jnp.float32)
        o_ref[...] = h2 * nout

    return pl.pallas_call(
        body,
        grid=(N // BN_ROWS,),
        in_specs=[
            pl.BlockSpec((BN_ROWS, H), lambda i: (i, 0)),
            pl.BlockSpec((8, H), lambda i: (0, 0)),
            pl.BlockSpec((1, H), lambda i: (0, 0)),
            pl.BlockSpec((1, H), lambda i: (0, 0)),
            pl.BlockSpec((H, CP), lambda i: (0, 0)),
            pl.BlockSpec((2, BN_ROWS, 16), lambda i: (0, i, 0)),
        ],
        out_specs=pl.BlockSpec((BN_ROWS, CP), lambda i: (i, 0)),
        out_shape=jax.ShapeDtypeStruct((N, CP), jnp.float32),
    )(t, stats, gamma1, beta1, W2p, degsum)


def _tc_final(agg_part, degsum, b2p):
    def body(a_ref, ds_ref, b_ref, o_ref):
        nin = _norm_from_deg(ds_ref[1][:, 0:1])
        u = (a_ref[0] + a_ref[1]) * nin + b_ref[...]
        col = lax.broadcasted_iota(jnp.int32, (BN_ROWS, CP), 1)
        valid = col < C
        um = jnp.where(valid, u, -1e30)
        m = jnp.max(um, axis=1, keepdims=True)
        e = jnp.where(valid, jnp.exp(um - m), 0.0)
        ssum = jnp.sum(e, axis=1, keepdims=True)
        o_ref[...] = u - m - jnp.log(ssum)

    return pl.pallas_call(
        body,
        grid=(N // BN_ROWS,),
        in_specs=[
            pl.BlockSpec((NC, BN_ROWS, CP), lambda i: (0, i, 0)),
            pl.BlockSpec((2, BN_ROWS, 16), lambda i: (0, i, 0)),
            pl.BlockSpec((1, CP), lambda i: (0, 0)),
        ],
        out_specs=pl.BlockSpec((BN_ROWS, CP), lambda i: (i, 0)),
        out_shape=jax.ShapeDtypeStruct((N, CP), jnp.float32),
    )(agg_part, degsum, b2p)


def kernel(x, edge_index, W1, b1, gamma1, beta1, W2, b2):
    src = edge_index[0]
    dst = edge_index[1]

    deg_part = _sc_degrees(src, dst)
    h1, degsum = _tc_layer1_pre(x, W1, deg_part)
    agg1 = _sc_aggregate(h1, src, dst, H)
    t, stats = _tc_combine_stats(agg1, degsum, b1.reshape(1, H))

    W2p = jnp.pad(W2, ((0, 0), (0, CP - C)))
    b2p = jnp.pad(b2, (0, CP - C)).reshape(1, CP)
    h2 = _tc_bn_relu_mm(t, stats, gamma1.reshape(1, H), beta1.reshape(1, H),
                        W2p, degsum)
    agg2 = _sc_aggregate(h2, src, dst, CP)
    out = _tc_final(agg2, degsum, b2p)
    return out[:, :C]


# speedup vs baseline: 10.3516x; 2.0914x over previous
import functools

import jax
import jax.numpy as jnp
from jax import lax
from jax.experimental import pallas as pl
from jax.experimental.pallas import tpu as pltpu
from jax.experimental.pallas import tpu_sc as plsc

N = 10000
E = 320000
D = 128
H = 128
C = 40
CP = 48
BN_EPS = 1e-5

NC = 2
NS = 16
K = 100
EPT = E // (NC * NS)
NCHUNK = EPT // K
RPT = 624
TAIL = N - NS * RPT


def _zero_fill(ref, nrows, width):
    zeros16 = jnp.zeros((16,), jnp.float32)

    @pl.loop(0, nrows)
    def _(r):
        for g in range(width // 16):
            ref[r, pl.ds(g * 16, 16)] = zeros16


def _zero_shared(sh, zbuf, s, width):
    for j in range(RPT // K):
        pltpu.sync_copy(zbuf, sh.at[pl.ds(s * RPT + j * K, K)])
    rem = RPT % K
    if rem:
        pltpu.sync_copy(zbuf.at[pl.ds(0, rem)],
                        sh.at[pl.ds(s * RPT + (RPT // K) * K, rem)])

    @pl.when(s == 0)
    def _():
        pltpu.sync_copy(zbuf.at[pl.ds(0, TAIL)],
                        sh.at[pl.ds(NS * RPT, TAIL)])


def _sc_degrees(src, dst):
    mesh = plsc.VectorSubcoreMesh(core_axis_name="c", subcore_axis_name="s")

    @functools.partial(
        pl.kernel,
        out_type=jax.ShapeDtypeStruct((NC, 2, N, 16), jnp.float32),
        mesh=mesh,
        compiler_params=pltpu.CompilerParams(use_tc_tiling_on_sc=False),
        scratch_types=[
            pltpu.VMEM((NCHUNK, K), jnp.int32),
            pltpu.VMEM((NCHUNK, K), jnp.int32),
            pltpu.VMEM((K, 16), jnp.float32),
            pltpu.VMEM((K, 16), jnp.float32),
            pltpu.VMEM_SHARED((N, 16), jnp.float32),
            pltpu.VMEM_SHARED((N, 16), jnp.float32),
            pltpu.SemaphoreType.DMA,
            pltpu.SemaphoreType.DMA,
        ],
    )
    def deg_kernel(src_hbm, dst_hbm, out_hbm, idx_s, idx_d, ones_v, zbuf,
                   dout_sh, din_sh, so_sem, si_sem):
        c = lax.axis_index("c")
        s = lax.axis_index("s")

        @pl.loop(0, K)
        def _(r):
            ones_v[r, pl.ds(0, 16)] = jnp.ones((16,), jnp.float32)

        _zero_fill(zbuf, K, 16)
        _zero_shared(dout_sh, zbuf, s, 16)
        _zero_shared(din_sh, zbuf, s, 16)
        plsc.subcore_barrier()

        wid = c * NS + s
        cbase = wid * NCHUNK
        pltpu.sync_copy(src_hbm.at[pl.ds(cbase, NCHUNK)], idx_s)
        pltpu.sync_copy(dst_hbm.at[pl.ds(cbase, NCHUNK)], idx_d)

        LAG = 8

        @pl.loop(0, NCHUNK)
        def _(i):
            pltpu.async_copy(ones_v, dout_sh.at[idx_s.at[i]], so_sem,
                             add=True)
            pltpu.async_copy(ones_v, din_sh.at[idx_d.at[i]], si_sem,
                             add=True)

            @pl.when(i >= LAG)
            def _():
                il = jnp.maximum(i - LAG, 0)
                pltpu.make_async_copy(ones_v, dout_sh.at[idx_s.at[il]],
                                      so_sem).wait()
                pltpu.make_async_copy(ones_v, din_sh.at[idx_d.at[il]],
                                      si_sem).wait()

        @pl.loop(NCHUNK - LAG, NCHUNK)
        def _(i):
            pltpu.make_async_copy(ones_v, dout_sh.at[idx_s.at[i]],
                                  so_sem).wait()
            pltpu.make_async_copy(ones_v, din_sh.at[idx_d.at[i]],
                                  si_sem).wait()

        plsc.subcore_barrier()
        rbase = s * RPT
        pltpu.sync_copy(dout_sh.at[pl.ds(rbase, RPT)],
                        out_hbm.at[c, 0, pl.ds(rbase, RPT)])
        pltpu.sync_copy(din_sh.at[pl.ds(rbase, RPT)],
                        out_hbm.at[c, 1, pl.ds(rbase, RPT)])

        @pl.when(s == 0)
        def _():
            pltpu.sync_copy(dout_sh.at[pl.ds(NS * RPT, TAIL)],
                            out_hbm.at[c, 0, pl.ds(NS * RPT, TAIL)])
            pltpu.sync_copy(din_sh.at[pl.ds(NS * RPT, TAIL)],
                            out_hbm.at[c, 1, pl.ds(NS * RPT, TAIL)])

    return deg_kernel(src, dst)


def _sc_aggregate(h, src2d, dst2d, width):
    mesh = plsc.VectorSubcoreMesh(core_axis_name="c", subcore_axis_name="s")

    @functools.partial(
        pl.kernel,
        out_type=jax.ShapeDtypeStruct((NC, N, width), jnp.float32),
        mesh=mesh,
        compiler_params=pltpu.CompilerParams(use_tc_tiling_on_sc=False),
        scratch_types=[
            pltpu.VMEM((NCHUNK, K), jnp.int32),
            pltpu.VMEM((NCHUNK, K), jnp.int32),
            pltpu.VMEM((K, width), jnp.float32),
            pltpu.VMEM((K, width), jnp.float32),
            pltpu.VMEM_SHARED((N, width), jnp.float32),
            pltpu.SemaphoreType.DMA,
            pltpu.SemaphoreType.DMA,
            pltpu.SemaphoreType.DMA,
            pltpu.SemaphoreType.DMA,
        ],
    )
    def agg_kernel(h_hbm, src_hbm, dst_hbm, out_hbm, idx_s, idx_d,
                   r0, r1, acc_sh, g0, g1, s0, s1):
        c = lax.axis_index("c")
        s = lax.axis_index("s")
        rows = [r0, r1]
        gsem = [g0, g1]
        ssem = [s0, s1]

        _zero_fill(r0, K, width)
        _zero_shared(acc_sh, r0, s, width)
        plsc.subcore_barrier()

        wid = c * NS + s
        cbase = wid * NCHUNK
        pltpu.sync_copy(src_hbm.at[pl.ds(cbase, NCHUNK)], idx_s)
        pltpu.sync_copy(dst_hbm.at[pl.ds(cbase, NCHUNK)], idx_d)

        pltpu.async_copy(h_hbm.at[idx_s.at[0]], rows[0], gsem[0])

        @pl.loop(0, NCHUNK // 2)
        def _(j):
            for b in range(2):
                i = j * 2 + b
                bn = 1 - b
                pltpu.make_async_copy(h_hbm.at[idx_s.at[i]], rows[b],
                                      gsem[b]).wait()
                pltpu.async_copy(rows[b], acc_sh.at[idx_d.at[i]], ssem[b],
                                 add=True)

                @pl.when(i >= 1)
                def _():
                    ip = jnp.maximum(i - 1, 0)
                    pltpu.make_async_copy(rows[bn], acc_sh.at[idx_d.at[ip]],
                                          ssem[bn]).wait()

                @pl.when(i + 1 < NCHUNK)
                def _():
                    i1 = jnp.minimum(i + 1, NCHUNK - 1)
                    pltpu.async_copy(h_hbm.at[idx_s.at[i1]], rows[bn],
                                     gsem[bn])

        pltpu.make_async_copy(rows[1], acc_sh.at[idx_d.at[NCHUNK - 1]],
                              ssem[1]).wait()

        plsc.subcore_barrier()
        rbase = s * RPT
        pltpu.sync_copy(acc_sh.at[pl.ds(rbase, RPT)],
                        out_hbm.at[c, pl.ds(rbase, RPT)])

        @pl.when(s == 0)
        def _():
            pltpu.sync_copy(acc_sh.at[pl.ds(NS * RPT, TAIL)],
                            out_hbm.at[c, pl.ds(NS * RPT, TAIL)])

    return agg_kernel(h, src2d, dst2d)


def _norm_from_deg(deg):
    return jnp.where(deg > 0.0, lax.rsqrt(jnp.maximum(deg, 1e-30)), 0.0)


BN_ROWS = 1000


def _tc_layer1_pre(x, W1, deg_part):
    def body(x_ref, w_ref, dp_ref, h_ref, ds_ref):
        dp = dp_ref[...]
        dsum = dp[0] + dp[1]
        ds_ref[...] = dsum
        nout = _norm_from_deg(dsum[0][:, 0:1])
        h = jnp.dot(x_ref[...], w_ref[...], preferred_element_type=jnp.float32)
        h_ref[...] = h * nout

    return pl.pallas_call(
        body,
        grid=(N // BN_ROWS,),
        in_specs=[
            pl.BlockSpec((BN_ROWS, D), lambda i: (i, 0)),
            pl.BlockSpec((D, H), lambda i: (0, 0)),
            pl.BlockSpec((NC, 2, BN_ROWS, 16), lambda i: (0, 0, i, 0)),
        ],
        out_specs=[
            pl.BlockSpec((BN_ROWS, H), lambda i: (i, 0)),
            pl.BlockSpec((2, BN_ROWS, 16), lambda i: (0, i, 0)),
        ],
        out_shape=[
            jax.ShapeDtypeStruct((N, H), jnp.float32),
            jax.ShapeDtypeStruct((2, N, 16), jnp.float32),
        ],
    )(x, W1, deg_part)


def _tc_combine_stats(agg_part, degsum, b1):
    def body(a_ref, ds_ref, b_ref, t_ref, st_ref):
        i = pl.program_id(0)
        nin = _norm_from_deg(ds_ref[1][:, 0:1])
        t = (a_ref[0] + a_ref[1]) * nin + b_ref[...]
        t_ref[...] = t

        @pl.when(i == 0)
        def _():
            st_ref[...] = jnp.zeros_like(st_ref)

        st_ref[0:1, :] += jnp.sum(t, axis=0, keepdims=True)
        st_ref[1:2, :] += jnp.sum(t * t, axis=0, keepdims=True)

    return pl.pallas_call(
        body,
        grid=(N // BN_ROWS,),
        in_specs=[
            pl.BlockSpec((NC, BN_ROWS, H), lambda i: (0, i, 0)),
            pl.BlockSpec((2, BN_ROWS, 16), lambda i: (0, i, 0)),
            pl.BlockSpec((1, H), lambda i: (0, 0)),
        ],
        out_specs=[
            pl.BlockSpec((BN_ROWS, H), lambda i: (i, 0)),
            pl.BlockSpec((8, H), lambda i: (0, 0)),
        ],
        out_shape=[
            jax.ShapeDtypeStruct((N, H), jnp.float32),
            jax.ShapeDtypeStruct((8, H), jnp.float32),
        ],
    )(agg_part, degsum, b1)


def _tc_bn_relu_mm(t, stats, gamma1, beta1, W2p, degsum):
    def body(t_ref, st_ref, g_ref, be_ref, w_ref, ds_ref, o_ref):
        inv_n = 1.0 / N
        mean = st_ref[0:1, :] * inv_n
        var = st_ref[1:2, :] * inv_n - mean * mean
        scale = g_ref[...] * lax.rsqrt(var + BN_EPS)
        z = (t_ref[...] - mean) * scale + be_ref[...]
        z = jnp.maximum(z, 0.0)
        nout = _norm_from_deg(ds_ref[0][:, 0:1])
        h2 = jnp.dot(z, w_ref[...], preferred_element_type=jnp.float32)
        o_ref[...] = h2 * nout

    return pl.pallas_call(
        body,
        grid=(N // BN_ROWS,),
        in_specs=[
            pl.BlockSpec((BN_ROWS, H), lambda i: (i, 0)),
            pl.BlockSpec((8, H), lambda i: (0, 0)),
            pl.BlockSpec((1, H), lambda i: (0, 0)),
            pl.BlockSpec((1, H), lambda i: (0, 0)),
            pl.BlockSpec((H, CP), lambda i: (0, 0)),
            pl.BlockSpec((2, BN_ROWS, 16), lambda i: (0, i, 0)),
        ],
        out_specs=pl.BlockSpec((BN_ROWS, CP), lambda i: (i, 0)),
        out_shape=jax.ShapeDtypeStruct((N, CP), jnp.float32),
    )(t, stats, gamma1, beta1, W2p, degsum)


def _tc_final(agg_part, degsum, b2p):
    def body(a_ref, ds_ref, b_ref, o_ref):
        nin = _norm_from_deg(ds_ref[1][:, 0:1])
        u = (a_ref[0] + a_ref[1]) * nin + b_ref[...]
        col = lax.broadcasted_iota(jnp.int32, (BN_ROWS, CP), 1)
        valid = col < C
        um = jnp.where(valid, u, -1e30)
        m = jnp.max(um, axis=1, keepdims=True)
        e = jnp.where(valid, jnp.exp(um - m), 0.0)
        ssum = jnp.sum(e, axis=1, keepdims=True)
        o_ref[...] = u - m - jnp.log(ssum)

    return pl.pallas_call(
        body,
        grid=(N // BN_ROWS,),
        in_specs=[
            pl.BlockSpec((NC, BN_ROWS, CP), lambda i: (0, i, 0)),
            pl.BlockSpec((2, BN_ROWS, 16), lambda i: (0, i, 0)),
            pl.BlockSpec((1, CP), lambda i: (0, 0)),
        ],
        out_specs=pl.BlockSpec((BN_ROWS, CP), lambda i: (i, 0)),
        out_shape=jax.ShapeDtypeStruct((N, CP), jnp.float32),
    )(agg_part, degsum, b2p)


def kernel(x, edge_index, W1, b1, gamma1, beta1, W2, b2):
    src = edge_index[0].reshape(E // K, K)
    dst = edge_index[1].reshape(E // K, K)

    deg_part = _sc_degrees(src, dst)
    h1, degsum = _tc_layer1_pre(x, W1, deg_part)
    agg1 = _sc_aggregate(h1, src, dst, H)
    t, stats = _tc_combine_stats(agg1, degsum, b1.reshape(1, H))

    W2p = jnp.pad(W2, ((0, 0), (0, CP - C)))
    b2p = jnp.pad(b2, (0, CP - C)).reshape(1, CP)
    h2 = _tc_bn_relu_mm(t, stats, gamma1.reshape(1, H), beta1.reshape(1, H),
                        W2p, degsum)
    agg2 = _sc_aggregate(h2, src, dst, CP)
    out = _tc_final(agg2, degsum, b2p)
    return out[:, :C]


# speedup vs baseline: 11.4195x; 1.1032x over previous
import functools

import jax
import jax.numpy as jnp
from jax import lax
from jax.experimental import pallas as pl
from jax.experimental.pallas import tpu as pltpu
from jax.experimental.pallas import tpu_sc as plsc

N = 10000
E = 320000
D = 128
H = 128
C = 40
CP = 48
BN_EPS = 1e-5

NC = 2
NS = 16
K = 100
EPT = E // (NC * NS)
NCHUNK = EPT // K
RPT = 624
TAIL = N - NS * RPT


def _sc_degrees(src, dst, z16):
    mesh = plsc.VectorSubcoreMesh(core_axis_name="c", subcore_axis_name="s")

    @functools.partial(
        pl.kernel,
        out_type=jax.ShapeDtypeStruct((NC, 2, N, 16), jnp.float32),
        mesh=mesh,
        compiler_params=pltpu.CompilerParams(use_tc_tiling_on_sc=False),
        scratch_types=[
            pltpu.VMEM((NCHUNK, K), jnp.int32),
            pltpu.VMEM((NCHUNK, K), jnp.int32),
            pltpu.VMEM((K, 16), jnp.float32),
            pltpu.VMEM_SHARED((N, 16), jnp.float32),
            pltpu.VMEM_SHARED((N, 16), jnp.float32),
            pltpu.SemaphoreType.DMA,
            pltpu.SemaphoreType.DMA,
            pltpu.SemaphoreType.DMA,
        ],
    )
    def deg_kernel(src_hbm, dst_hbm, z_hbm, out_hbm, idx_s, idx_d, ones_v,
                   dout_sh, din_sh, so_sem, si_sem, p_sem):
        c = lax.axis_index("c")
        s = lax.axis_index("s")
        wid = c * NS + s
        cbase = wid * NCHUNK
        rbase = s * RPT

        cps = [
            pltpu.make_async_copy(z_hbm.at[pl.ds(rbase, RPT)],
                                  dout_sh.at[pl.ds(rbase, RPT)], p_sem),
            pltpu.make_async_copy(z_hbm.at[pl.ds(rbase, RPT)],
                                  din_sh.at[pl.ds(rbase, RPT)], p_sem),
            pltpu.make_async_copy(src_hbm.at[pl.ds(cbase, NCHUNK)], idx_s,
                                  p_sem),
            pltpu.make_async_copy(dst_hbm.at[pl.ds(cbase, NCHUNK)], idx_d,
                                  p_sem),
        ]
        for cp in cps:
            cp.start()

        @pl.loop(0, K)
        def _(r):
            ones_v[r, pl.ds(0, 16)] = jnp.ones((16,), jnp.float32)

        @pl.when(s == 0)
        def _():
            pltpu.sync_copy(z_hbm.at[pl.ds(NS * RPT, TAIL)],
                            dout_sh.at[pl.ds(NS * RPT, TAIL)])
            pltpu.sync_copy(z_hbm.at[pl.ds(NS * RPT, TAIL)],
                            din_sh.at[pl.ds(NS * RPT, TAIL)])

        for cp in cps:
            cp.wait()
        plsc.subcore_barrier()

        LAG = 8

        @pl.loop(0, NCHUNK)
        def _(i):
            pltpu.async_copy(ones_v, dout_sh.at[idx_s.at[i]], so_sem,
                             add=True)
            pltpu.async_copy(ones_v, din_sh.at[idx_d.at[i]], si_sem,
                             add=True)

            @pl.when(i >= LAG)
            def _():
                il = jnp.maximum(i - LAG, 0)
                pltpu.make_async_copy(ones_v, dout_sh.at[idx_s.at[il]],
                                      so_sem).wait()
                pltpu.make_async_copy(ones_v, din_sh.at[idx_d.at[il]],
                                      si_sem).wait()

        @pl.loop(NCHUNK - LAG, NCHUNK)
        def _(i):
            pltpu.make_async_copy(ones_v, dout_sh.at[idx_s.at[i]],
                                  so_sem).wait()
            pltpu.make_async_copy(ones_v, din_sh.at[idx_d.at[i]],
                                  si_sem).wait()

        plsc.subcore_barrier()
        pltpu.sync_copy(dout_sh.at[pl.ds(rbase, RPT)],
                        out_hbm.at[c, 0, pl.ds(rbase, RPT)])
        pltpu.sync_copy(din_sh.at[pl.ds(rbase, RPT)],
                        out_hbm.at[c, 1, pl.ds(rbase, RPT)])

        @pl.when(s == 0)
        def _():
            pltpu.sync_copy(dout_sh.at[pl.ds(NS * RPT, TAIL)],
                            out_hbm.at[c, 0, pl.ds(NS * RPT, TAIL)])
            pltpu.sync_copy(din_sh.at[pl.ds(NS * RPT, TAIL)],
                            out_hbm.at[c, 1, pl.ds(NS * RPT, TAIL)])

    return deg_kernel(src, dst, z16)


def _make_sc_aggregate(width, nb, gla):
    mesh = plsc.VectorSubcoreMesh(core_axis_name="c", subcore_axis_name="s")

    @functools.partial(
        pl.kernel,
        out_type=jax.ShapeDtypeStruct((NC, N, width), jnp.float32),
        mesh=mesh,
        compiler_params=pltpu.CompilerParams(use_tc_tiling_on_sc=False),
        scratch_types=(
            [pltpu.VMEM((NCHUNK, K), jnp.int32),
             pltpu.VMEM((NCHUNK, K), jnp.int32)]
            + [pltpu.VMEM((K, width), jnp.float32) for _ in range(nb)]
            + [pltpu.VMEM_SHARED((N, width), jnp.float32)]
            + [pltpu.SemaphoreType.DMA for _ in range(2 * nb + 1)]
        ),
    )
    def agg_kernel(h_hbm, src_hbm, dst_hbm, z_hbm, out_hbm, idx_s, idx_d,
                   *bufs):
        rows = bufs[:nb]
        acc_sh = bufs[nb]
        gsem = bufs[nb + 1:nb + 1 + nb]
        ssem = bufs[nb + 1 + nb:nb + 1 + 2 * nb]
        p_sem = bufs[2 * nb + 1 + nb]
        c = lax.axis_index("c")
        s = lax.axis_index("s")
        wid = c * NS + s
        cbase = wid * NCHUNK
        rbase = s * RPT

        cps = [
            pltpu.make_async_copy(z_hbm.at[pl.ds(rbase, RPT)],
                                  acc_sh.at[pl.ds(rbase, RPT)], p_sem),
            pltpu.make_async_copy(src_hbm.at[pl.ds(cbase, NCHUNK)], idx_s,
                                  p_sem),
            pltpu.make_async_copy(dst_hbm.at[pl.ds(cbase, NCHUNK)], idx_d,
                                  p_sem),
        ]
        for cp in cps:
            cp.start()

        @pl.when(s == 0)
        def _():
            pltpu.sync_copy(z_hbm.at[pl.ds(NS * RPT, TAIL)],
                            acc_sh.at[pl.ds(NS * RPT, TAIL)])

        for cp in cps:
            cp.wait()
        plsc.subcore_barrier()

        for g in range(gla):
            pltpu.async_copy(h_hbm.at[idx_s.at[g]], rows[g], gsem[g])

        @pl.loop(0, NCHUNK // nb)
        def _(j):
            for b in range(nb):
                i = j * nb + b
                bn = (b + gla) % nb
                pltpu.make_async_copy(h_hbm.at[idx_s.at[i]], rows[b],
                                      gsem[b]).wait()
                pltpu.async_copy(rows[b], acc_sh.at[idx_d.at[i]], ssem[b],
                                 add=True)

                @pl.when(i + gla - nb >= 0)
                def _():
                    ip = jnp.maximum(i + gla - nb, 0)
                    pltpu.make_async_copy(rows[bn], acc_sh.at[idx_d.at[ip]],
                                          ssem[bn]).wait()

                @pl.when(i + gla < NCHUNK)
                def _():
                    i2 = jnp.minimum(i + gla, NCHUNK - 1)
                    pltpu.async_copy(h_hbm.at[idx_s.at[i2]], rows[bn],
                                     gsem[bn])

        for t in range(nb - gla):
            ic = NCHUNK - (nb - gla) + t
            pltpu.make_async_copy(rows[ic % nb], acc_sh.at[idx_d.at[ic]],
                                  ssem[ic % nb]).wait()

        plsc.subcore_barrier()
        pltpu.sync_copy(acc_sh.at[pl.ds(rbase, RPT)],
                        out_hbm.at[c, pl.ds(rbase, RPT)])

        @pl.when(s == 0)
        def _():
            pltpu.sync_copy(acc_sh.at[pl.ds(NS * RPT, TAIL)],
                            out_hbm.at[c, pl.ds(NS * RPT, TAIL)])

    return agg_kernel


_agg_h = _make_sc_aggregate(H, 2, 1)
_agg_cp = _make_sc_aggregate(CP, 4, 2)


def _norm_from_deg(deg):
    return jnp.where(deg > 0.0, lax.rsqrt(jnp.maximum(deg, 1e-30)), 0.0)


BN_ROWS = 1000


def _tc_mm(x, W1):
    def body(x_ref, w_ref, o_ref):
        o_ref[...] = jnp.dot(x_ref[...], w_ref[...],
                             preferred_element_type=jnp.float32)

    return pl.pallas_call(
        body,
        grid=(N // BN_ROWS,),
        in_specs=[
            pl.BlockSpec((BN_ROWS, D), lambda i: (i, 0)),
            pl.BlockSpec((D, H), lambda i: (0, 0)),
        ],
        out_specs=pl.BlockSpec((BN_ROWS, H), lambda i: (i, 0)),
        out_shape=jax.ShapeDtypeStruct((N, H), jnp.float32),
    )(x, W1)


def _tc_scale(xw, deg_part):
    def body(xw_ref, dp_ref, h_ref, ds_ref):
        dp = dp_ref[...]
        dsum = dp[0] + dp[1]
        ds_ref[...] = dsum
        nout = _norm_from_deg(dsum[0][:, 0:1])
        h_ref[...] = xw_ref[...] * nout

    return pl.pallas_call(
        body,
        grid=(N // BN_ROWS,),
        in_specs=[
            pl.BlockSpec((BN_ROWS, H), lambda i: (i, 0)),
            pl.BlockSpec((NC, 2, BN_ROWS, 16), lambda i: (0, 0, i, 0)),
        ],
        out_specs=[
            pl.BlockSpec((BN_ROWS, H), lambda i: (i, 0)),
            pl.BlockSpec((2, BN_ROWS, 16), lambda i: (0, i, 0)),
        ],
        out_shape=[
            jax.ShapeDtypeStruct((N, H), jnp.float32),
            jax.ShapeDtypeStruct((2, N, 16), jnp.float32),
        ],
    )(xw, deg_part)


def _tc_combine_stats(agg_part, degsum, b1):
    def body(a_ref, ds_ref, b_ref, t_ref, st_ref):
        i = pl.program_id(0)
        nin = _norm_from_deg(ds_ref[1][:, 0:1])
        t = (a_ref[0] + a_ref[1]) * nin + b_ref[...]
        t_ref[...] = t

        @pl.when(i == 0)
        def _():
            st_ref[...] = jnp.zeros_like(st_ref)

        st_ref[0:1, :] += jnp.sum(t, axis=0, keepdims=True)
        st_ref[1:2, :] += jnp.sum(t * t, axis=0, keepdims=True)

    return pl.pallas_call(
        body,
        grid=(N // BN_ROWS,),
        in_specs=[
            pl.BlockSpec((NC, BN_ROWS, H), lambda i: (0, i, 0)),
            pl.BlockSpec((2, BN_ROWS, 16), lambda i: (0, i, 0)),
            pl.BlockSpec((1, H), lambda i: (0, 0)),
        ],
        out_specs=[
            pl.BlockSpec((BN_ROWS, H), lambda i: (i, 0)),
            pl.BlockSpec((8, H), lambda i: (0, 0)),
        ],
        out_shape=[
            jax.ShapeDtypeStruct((N, H), jnp.float32),
            jax.ShapeDtypeStruct((8, H), jnp.float32),
        ],
    )(agg_part, degsum, b1)


def _tc_bn_relu_mm(t, stats, gamma1, beta1, W2p, degsum):
    def body(t_ref, st_ref, g_ref, be_ref, w_ref, ds_ref, o_ref):
        inv_n = 1.0 / N
        mean = st_ref[0:1, :] * inv_n
        var = st_ref[1:2, :] * inv_n - mean * mean
        scale = g_ref[...] * lax.rsqrt(var + BN_EPS)
        z = (t_ref[...] - mean) * scale + be_ref[...]
        z = jnp.maximum(z, 0.0)
        nout = _norm_from_deg(ds_ref[0][:, 0:1])
        h2 = jnp.dot(z, w_ref[...], preferred_element_type=jnp.float32)
        o_ref[...] = h2 * nout

    return pl.pallas_call(
        body,
        grid=(N // BN_ROWS,),
        in_specs=[
            pl.BlockSpec((BN_ROWS, H), lambda i: (i, 0)),
            pl.BlockSpec((8, H), lambda i: (0, 0)),
            pl.BlockSpec((1, H), lambda i: (0, 0)),
            pl.BlockSpec((1, H), lambda i: (0, 0)),
            pl.BlockSpec((H, CP), lambda i: (0, 0)),
            pl.BlockSpec((2, BN_ROWS, 16), lambda i: (0, i, 0)),
        ],
        out_specs=pl.BlockSpec((BN_ROWS, CP), lambda i: (i, 0)),
        out_shape=jax.ShapeDtypeStruct((N, CP), jnp.float32),
    )(t, stats, gamma1, beta1, W2p, degsum)


def _tc_final(agg_part, degsum, b2p):
    def body(a_ref, ds_ref, b_ref, o_ref):
        nin = _norm_from_deg(ds_ref[1][:, 0:1])
        u = (a_ref[0] + a_ref[1]) * nin + b_ref[...]
        col = lax.broadcasted_iota(jnp.int32, (BN_ROWS, CP), 1)
        valid = col < C
        um = jnp.where(valid, u, -1e30)
        m = jnp.max(um, axis=1, keepdims=True)
        e = jnp.where(valid, jnp.exp(um - m), 0.0)
        ssum = jnp.sum(e, axis=1, keepdims=True)
        o_ref[...] = u - m - jnp.log(ssum)

    return pl.pallas_call(
        body,
        grid=(N // BN_ROWS,),
        in_specs=[
            pl.BlockSpec((NC, BN_ROWS, CP), lambda i: (0, i, 0)),
            pl.BlockSpec((2, BN_ROWS, 16), lambda i: (0, i, 0)),
            pl.BlockSpec((1, CP), lambda i: (0, 0)),
        ],
        out_specs=pl.BlockSpec((BN_ROWS, CP), lambda i: (i, 0)),
        out_shape=jax.ShapeDtypeStruct((N, CP), jnp.float32),
    )(agg_part, degsum, b2p)


def kernel(x, edge_index, W1, b1, gamma1, beta1, W2, b2):
    src = edge_index[0].reshape(E // K, K)
    dst = edge_index[1].reshape(E // K, K)
    z16 = jnp.zeros((N, 16), jnp.float32)
    zH = jnp.zeros((N, H), jnp.float32)
    zCP = jnp.zeros((N, CP), jnp.float32)

    xw = _tc_mm(x, W1)
    deg_part = _sc_degrees(src, dst, z16)
    h1, degsum = _tc_scale(xw, deg_part)
    agg1 = _agg_h(h1, src, dst, zH)
    t, stats = _tc_combine_stats(agg1, degsum, b1.reshape(1, H))

    W2p = jnp.pad(W2, ((0, 0), (0, CP - C)))
    b2p = jnp.pad(b2, (0, CP - C)).reshape(1, CP)
    h2 = _tc_bn_relu_mm(t, stats, gamma1.reshape(1, H), beta1.reshape(1, H),
                        W2p, degsum)
    agg2 = _agg_cp(h2, src, dst, zCP)
    out = _tc_final(agg2, degsum, b2p)
    return out[:, :C]
